# Initial kernel scaffold; baseline (speedup 1.0000x reference)
import jax
import jax.numpy as jnp
from jax.experimental import pallas as pl


def kernel(x, adj, H_, params):
    raise NotImplementedError("write your pallas kernel here")


# speedup vs baseline: 19.5600x; 19.5600x over previous
import functools

import jax
import jax.numpy as jnp
from jax import lax
from jax.experimental import pallas as pl
from jax.experimental.pallas import tpu as pltpu
from jax.experimental.pallas import tpu_sc as plsc

_MODENUM = 2
_NODENUM = 2500
_BATCH = 4
_SLID = 128
_N = _NODENUM * _BATCH
_E = 320000
_EMB = 4
_HEADS = 6
_GRUH = 16

_C = 256
_SUB = 128
_NSUB = _C // _SUB
_EPAD = 327680
_NROW = _N + 8
_STRIPE = 624


def _leaky(x):
    return jnp.where(x >= 0, x, 0.2 * x)


@functools.lru_cache(maxsize=None)
def _make_edge_kernel(K, NH, HBMULT, esplit):
    vph = max(K // 16 // NH, 1)
    nchunks = _EPAD // 32 // _C if esplit else _EPAD // 16 // _C
    mesh = plsc.VectorSubcoreMesh(core_axis_name="c", subcore_axis_name="s")

    @functools.partial(
        pl.kernel,
        out_type=jax.ShapeDtypeStruct((2 * _N, 128), jnp.float32),
        mesh=mesh,
        scratch_types=[
            pltpu.VMEM((_NSUB, _SUB), jnp.int32),
            pltpu.VMEM((_NSUB, _SUB), jnp.int32),
            pltpu.VMEM((_C, 16), jnp.float32),
            pltpu.VMEM((_C, 16), jnp.float32),
            pltpu.VMEM((_C, 128), jnp.float32),
            pltpu.VMEM_SHARED((_NROW, 128), jnp.float32),
            pltpu.SemaphoreType.DMA,
        ],
        compiler_params=pltpu.CompilerParams(use_tc_tiling_on_sc=False),
    )
    def ek(src_hbm, dst_hbm, als_hbm, ald_hbm, tab0_hbm, tab1_hbm,
           znum_hbm, num_out,
           src_v, dst_v, als_v, ald_v, h_v, sh_num, sem):
        c = lax.axis_index("c")
        s = lax.axis_index("s")

        @pl.when(s < 15)
        def _():
            pltpu.sync_copy(znum_hbm.at[pl.ds(0, _STRIPE)],
                            sh_num.at[pl.ds(s * _STRIPE, _STRIPE)])

        @pl.when(s == 15)
        def _():
            pltpu.sync_copy(znum_hbm, sh_num.at[pl.ds(15 * _STRIPE, 648)])

        plsc.subcore_barrier()

        if esplit:
            ck0 = (s * 2 + c) * nchunks
        else:
            ck0 = s * nchunks

        def chunk_body(i, carry0):
            ck = ck0 + i
            pltpu.sync_copy(src_hbm.at[ck], src_v)
            pltpu.sync_copy(dst_hbm.at[ck], dst_v)
            for j in range(_NSUB):
                dst_sl = pl.ds(j * _SUB, _SUB)
                pltpu.async_copy(als_hbm.at[src_v.at[j]],
                                 als_v.at[dst_sl], sem).wait()
                pltpu.async_copy(ald_hbm.at[dst_v.at[j]],
                                 ald_v.at[dst_sl], sem).wait()

                @pl.when(c == 0)
                def _():
                    pltpu.async_copy(tab0_hbm.at[src_v.at[j]],
                                     h_v.at[dst_sl], sem).wait()

                @pl.when(c == 1)
                def _():
                    pltpu.async_copy(tab1_hbm.at[src_v.at[j]],
                                     h_v.at[dst_sl], sem).wait()

            def do_edges(hb):
                lanes = jnp.arange(16, dtype=jnp.int32)
                headmask = jnp.where((lanes >= hb) & (lanes < hb + NH),
                                     jnp.float32(1.0), jnp.float32(0.0))

                def edge_body(e, carry):
                    eerow = jnp.exp(_leaky(als_v[e] + ald_v[e]))
                    h_v[e, pl.ds(96, 16)] = eerow * headmask
                    for hh in range(NH):
                        m = eerow[hb + hh]
                        for jj in range(vph):
                            sl = pl.ds(16 * (hh * vph + jj), 16)
                            h_v[e, sl] = h_v[e, sl] * m
                    return carry

                lax.fori_loop(0, _C, edge_body, 0)

            if HBMULT == 0:
                do_edges(0)
            else:
                @pl.when(c == 0)
                def _():
                    do_edges(0)

                @pl.when(c == 1)
                def _():
                    do_edges(HBMULT)
            for j in range(_NSUB):
                src_sl = pl.ds(j * _SUB, _SUB)
                pltpu.sync_copy(h_v.at[src_sl],
                                sh_num.at[dst_v.at[j]], add=True)
            return carry0

        lax.fori_loop(0, nchunks, chunk_body, 0)
        plsc.subcore_barrier()
        ob = c * _N + s * _STRIPE

        @pl.when(s < 15)
        def _():
            pltpu.sync_copy(sh_num.at[pl.ds(s * _STRIPE, _STRIPE)],
                            num_out.at[pl.ds(ob, _STRIPE)])

        @pl.when(s == 15)
        def _():
            pltpu.sync_copy(sh_num.at[pl.ds(15 * _STRIPE, 640)],
                            num_out.at[pl.ds(c * _N + 15 * _STRIPE, 640)])

    return ek


_BN = 1000


@functools.lru_cache(maxsize=None)
def _make_proj_kernel(kin, kt):
    def body(x_ref, w_ref, as_ref, ad_ref, rex_ref,
             h_ref, als_ref, ald_ref, snum_ref, sden_ref):
        h = jnp.dot(x_ref[...], w_ref[...],
                    preferred_element_type=jnp.float32)
        h_ref[...] = h
        als = jnp.dot(h, as_ref[...], preferred_element_type=jnp.float32)
        ald = jnp.dot(h, ad_ref[...], preferred_element_type=jnp.float32)
        als_ref[...] = als
        ald_ref[...] = ald
        e = als + ald
        ee = jnp.exp(jnp.where(e >= 0, e, 0.2 * e))
        lanemask = (lax.broadcasted_iota(jnp.int32, (1, 16), 1) <
                    _n_heads(kt)).astype(jnp.float32)
        ee = ee * lanemask
        sden_ref[...] = ee
        snum_ref[...] = h * jnp.dot(ee, rex_ref[...],
                                    preferred_element_type=jnp.float32)

    grid = _N // _BN
    return pl.pallas_call(
        body,
        grid=(grid,),
        in_specs=[
            pl.BlockSpec((_BN, kin), lambda i: (i, 0)),
            pl.BlockSpec((kin, kt), lambda i: (0, 0)),
            pl.BlockSpec((kt, 16), lambda i: (0, 0)),
            pl.BlockSpec((kt, 16), lambda i: (0, 0)),
            pl.BlockSpec((16, kt), lambda i: (0, 0)),
        ],
        out_specs=[
            pl.BlockSpec((_BN, kt), lambda i: (i, 0)),
            pl.BlockSpec((_BN, 16), lambda i: (i, 0)),
            pl.BlockSpec((_BN, 16), lambda i: (i, 0)),
            pl.BlockSpec((_BN, kt), lambda i: (i, 0)),
            pl.BlockSpec((_BN, 16), lambda i: (i, 0)),
        ],
        out_shape=[
            jax.ShapeDtypeStruct((_N, kt), jnp.float32),
            jax.ShapeDtypeStruct((_N, 16), jnp.float32),
            jax.ShapeDtypeStruct((_N, 16), jnp.float32),
            jax.ShapeDtypeStruct((_N, kt), jnp.float32),
            jax.ShapeDtypeStruct((_N, 16), jnp.float32),
        ],
    )


def _n_heads(kt):
    return {192: 6, 4: 1, 128: 1}[kt]


@functools.lru_cache(maxsize=None)
def _make_finish_kernel(kt, esplit):
    heads = _n_heads(kt)

    def body(b0_ref, b1_ref, snum_ref, sden_ref, rex_ref, b_ref, o_ref):
        b0 = b0_ref[...]
        b1 = b1_ref[...]
        z = jnp.zeros((_BN, 16 - heads), jnp.float32)
        if esplit:
            num = b0[:, :kt] + b1[:, :kt]
            den16 = jnp.concatenate(
                [b0[:, 96:96 + 1] + b1[:, 96:96 + 1], z], axis=1)
        elif heads == 1:
            num = jnp.concatenate([b0[:, :kt // 2], b1[:, :kt // 2]], axis=1)
            den16 = jnp.concatenate([b0[:, 96:96 + 1], z], axis=1)
        else:
            nh = heads // 2
            num = jnp.concatenate([b0[:, :kt // 2], b1[:, :kt // 2]], axis=1)
            den16 = jnp.concatenate(
                [b0[:, 96:96 + nh], b1[:, 96 + nh:96 + 2 * nh], z], axis=1)
        den = jnp.dot(den16 + sden_ref[...], rex_ref[...],
                      preferred_element_type=jnp.float32)
        o_ref[...] = (num + snum_ref[...]) / (den + 1e-16) + b_ref[...]

    grid = _N // _BN
    return pl.pallas_call(
        body,
        grid=(grid,),
        in_specs=[
            pl.BlockSpec((_BN, 128), lambda i: (i, 0)),
            pl.BlockSpec((_BN, 128), lambda i: (i + _N // _BN, 0)),
            pl.BlockSpec((_BN, kt), lambda i: (i, 0)),
            pl.BlockSpec((_BN, 16), lambda i: (i, 0)),
            pl.BlockSpec((16, kt), lambda i: (0, 0)),
            pl.BlockSpec((1, kt), lambda i: (0, 0)),
        ],
        out_specs=pl.BlockSpec((_BN, kt), lambda i: (i, 0)),
        out_shape=jax.ShapeDtypeStruct((_N, kt), jnp.float32),
    )


@functools.lru_cache(maxsize=None)
def _make_mlp3_kernel(k0, k1, k2, k3):
    def body(x_ref, w1, b1, w2, b2, w3, b3, o_ref):
        z = jax.nn.relu(jnp.dot(x_ref[...], w1[...],
                                preferred_element_type=jnp.float32) + b1[...])
        z = jax.nn.relu(jnp.dot(z, w2[...],
                                preferred_element_type=jnp.float32) + b2[...])
        o_ref[...] = jnp.dot(z, w3[...],
                             preferred_element_type=jnp.float32) + b3[...]

    grid = _N // _BN
    return pl.pallas_call(
        body,
        grid=(grid,),
        in_specs=[
            pl.BlockSpec((_BN, k0), lambda i: (i, 0)),
            pl.BlockSpec((k0, k1), lambda i: (0, 0)),
            pl.BlockSpec((1, k1), lambda i: (0, 0)),
            pl.BlockSpec((k1, k2), lambda i: (0, 0)),
            pl.BlockSpec((1, k2), lambda i: (0, 0)),
            pl.BlockSpec((k2, k3), lambda i: (0, 0)),
            pl.BlockSpec((1, k3), lambda i: (0, 0)),
        ],
        out_specs=pl.BlockSpec((_BN, k3), lambda i: (i, 0)),
        out_shape=jax.ShapeDtypeStruct((_N, k3), jnp.float32),
    )


def _make_cat_kernel():
    def body(g0_ref, l0_ref, g1_ref, l1_ref, p0_ref, p1_ref, o_ref):
        c0 = jnp.concatenate([g0_ref[...], l0_ref[...]], axis=1)
        c1 = jnp.concatenate([g1_ref[...], l1_ref[...]], axis=1)
        o_ref[...] = (
            jnp.dot(c0, p0_ref[...], preferred_element_type=jnp.float32) +
            jnp.dot(c1, p1_ref[...], preferred_element_type=jnp.float32))

    grid = _N // _BN
    return pl.pallas_call(
        body,
        grid=(grid,),
        in_specs=[
            pl.BlockSpec((_BN, 4), lambda i: (i, 0)),
            pl.BlockSpec((_BN, 4), lambda i: (i, 0)),
            pl.BlockSpec((_BN, 4), lambda i: (i, 0)),
            pl.BlockSpec((_BN, 4), lambda i: (i, 0)),
            pl.BlockSpec((8, 8), lambda i: (0, 0)),
            pl.BlockSpec((8, 8), lambda i: (0, 0)),
        ],
        out_specs=pl.BlockSpec((_BN, 8), lambda i: (i, 0)),
        out_shape=jax.ShapeDtypeStruct((_N, 8), jnp.float32),
    )


def _make_gru_kernel():
    T = _NODENUM

    def step(gi, h, whhT, bhh):
        gh = jnp.dot(h, whhT, preferred_element_type=jnp.float32) + bhh
        r = jax.nn.sigmoid(gi[:, 0:16] + gh[:, 0:16])
        z = jax.nn.sigmoid(gi[:, 16:32] + gh[:, 16:32])
        nn = jnp.tanh(gi[:, 32:48] + r * gh[:, 32:48])
        return (1.0 - z) * nn + z * h

    def body(cat_ref, h00_ref, h10_ref, wih0, whh0, bih0, bhh0,
             wih1, whh1, bih1, bhh1, y_ref, ht0_ref, ht1_ref, gi_ref):
        def fill(i, carry):
            o = pl.multiple_of(i * 200, 8)
            gi_ref[pl.ds(o, 200), :] = jnp.dot(
                cat_ref[pl.ds(o, 200), :], wih0[...],
                preferred_element_type=jnp.float32) + bih0[...]
            return carry

        lax.fori_loop(0, (4 * T) // 200, fill, 0)

        def loop(t2, hs):
            h0, h1 = hs
            o = pl.multiple_of(t2 * 8, 8)
            g2 = gi_ref[pl.ds(o, 8), :]
            ys = []
            for half in range(2):
                gi = g2[4 * half:4 * half + 4, :]
                h0 = step(gi, h0, whh0[...], bhh0[...])
                gi1 = jnp.dot(h0, wih1[...],
                              preferred_element_type=jnp.float32) + bih1[...]
                h1 = step(gi1, h1, whh1[...], bhh1[...])
                ys.append(h1)
            y_ref[pl.ds(o, 8), :] = jnp.concatenate(ys, axis=0)
            return (h0, h1)

        h0, h1 = lax.fori_loop(0, T // 2, loop,
                               (h00_ref[...], h10_ref[...]))
        ht0_ref[...] = h0
        ht1_ref[...] = h1

    return pl.pallas_call(
        body,
        out_shape=[
            jax.ShapeDtypeStruct((4 * T, _GRUH), jnp.float32),
            jax.ShapeDtypeStruct((_BATCH, _GRUH), jnp.float32),
            jax.ShapeDtypeStruct((_BATCH, _GRUH), jnp.float32),
        ],
        scratch_shapes=[pltpu.VMEM((4 * T, 48), jnp.float32)],
    )


def _make_cf_kernel():
    KB = 2048
    NK = 40960 // KB

    def body(x_ref, w1_ref, b1, w2, b2, w3, b3, o_ref, acc_ref):
        k = pl.program_id(0)

        @pl.when(k == 0)
        def _():
            acc_ref[...] = jnp.zeros_like(acc_ref)

        acc_ref[...] += jnp.dot(x_ref[...], w1_ref[...],
                                preferred_element_type=jnp.float32)

        @pl.when(k == NK - 1)
        def _():
            z = jax.nn.relu(acc_ref[...] + b1[...])
            z = jax.nn.relu(jnp.dot(z, w2[...],
                                    preferred_element_type=jnp.float32)
                            + b2[...])
            o_ref[...] = jax.nn.sigmoid(
                jnp.dot(z, w3[...], preferred_element_type=jnp.float32)
                + b3[...])

    return pl.pallas_call(
        body,
        grid=(NK,),
        in_specs=[
            pl.BlockSpec((_BATCH, KB), lambda k: (0, k)),
            pl.BlockSpec((KB, 512), lambda k: (k, 0)),
            pl.BlockSpec((1, 512), lambda k: (0, 0)),
            pl.BlockSpec((512, 64), lambda k: (0, 0)),
            pl.BlockSpec((1, 64), lambda k: (0, 0)),
            pl.BlockSpec((64, 2), lambda k: (0, 0)),
            pl.BlockSpec((1, 2), lambda k: (0, 0)),
        ],
        out_specs=pl.BlockSpec((_BATCH, 2), lambda k: (0, 0)),
        out_shape=jax.ShapeDtypeStruct((_BATCH, 2), jnp.float32),
        scratch_shapes=[pltpu.VMEM((_BATCH, 512), jnp.float32)],
    )


def _att_mat(a, kt):
    heads, outc = a.shape
    m = jnp.zeros((kt, 16), jnp.float32)
    for h in range(heads):
        m = m.at[h * outc:(h + 1) * outc, h].set(a[h])
    return m


def _rex_mat(kt):
    heads = _n_heads(kt)
    outc = kt // heads
    m = jnp.zeros((16, kt), jnp.float32)
    for h in range(heads):
        m = m.at[h, h * outc:(h + 1) * outc].set(1.0)
    return m


def _pad_rows(a, cols=None):
    cpad = 0 if cols is None else cols - a.shape[1]
    return jnp.pad(a, ((0, _NROW - _N), (0, cpad)))


def _gat_layer(xh, W, a_s, a_d, b, kt, src3, dst3, sc_cfg):
    K, NH, HBMULT, esplit = sc_cfg
    heads = _n_heads(kt)
    proj = _make_proj_kernel(xh.shape[1], kt)
    rex = _rex_mat(kt)
    h, als, ald, snum, sden = proj(xh, W, _att_mat(a_s, kt),
                                   _att_mat(a_d, kt), rex)
    if esplit:
        tab0 = tab1 = _pad_rows(h, 128)
    else:
        half = kt // 2
        tab0 = _pad_rows(h[:, :half], 128)
        tab1 = _pad_rows(h[:, half:], 128)
    ek = _make_edge_kernel(K, NH, HBMULT, esplit)
    znum = jnp.zeros((648, 128), jnp.float32)
    out = ek(src3, dst3, _pad_rows(als), _pad_rows(ald), tab0, tab1, znum)
    fin = _make_finish_kernel(kt, esplit)
    return fin(out, out, snum, sden, rex, b.reshape(1, kt))


_CFG_H6 = (96, 3, 3, False)
_CFG_R2 = (64, 1, 0, False)
_CFG_G2 = (16, 1, 0, True)


def kernel(x, adj, H_, params):
    p = params
    src3 = []
    dst3 = []
    for m in range(_MODENUM):
        sp = jnp.full((_EPAD,), _N, jnp.int32).at[:_E].set(adj[m, 0])
        dp = jnp.full((_EPAD,), _N, jnp.int32).at[:_E].set(adj[m, 1])
        src3.append(sp.reshape(-1, _NSUB, _SUB))
        dst3.append(dp.reshape(-1, _NSUB, _SUB))

    nf = _make_mlp3_kernel(_SLID, 256, 32, _EMB)
    cats = _make_cat_kernel()
    g_list = []
    lf_list = []
    for m in range(_MODENUM):
        mt = x[m * _N:(m + 1) * _N]
        g = _gat_layer(mt, p['g1_W'][m], p['g1_as'][m], p['g1_ad'][m],
                       p['g1_b'][m], 192, src3[m], dst3[m], _CFG_H6)
        g = _gat_layer(g, p['g2_W'][m], p['g2_as'][m], p['g2_ad'][m],
                       p['g2_b'][m], 4, src3[m], dst3[m], _CFG_G2)
        lf = nf(mt, p['nf_W1'], p['nf_b1'].reshape(1, -1),
                p['nf_W2'], p['nf_b2'].reshape(1, -1),
                p['nf_W3'], p['nf_b3'].reshape(1, -1))
        g_list.append(g)
        lf_list.append(lf)
    cat = cats(g_list[0], lf_list[0], g_list[1], lf_list[1],
               p['catP'][0], p['catP'][1])

    cat_tb = cat.reshape(_BATCH, _NODENUM, 8).transpose(1, 0, 2) \
        .reshape(_BATCH * _NODENUM, 8)
    gru = _make_gru_kernel()
    y_tb, h0T, h1T = gru(
        cat_tb, H_[0], H_[1],
        p['gru_Wih0'].T, p['gru_Whh0'].T,
        p['gru_bih0'].reshape(1, -1), p['gru_bhh0'].reshape(1, -1),
        p['gru_Wih1'].T, p['gru_Whh1'].T,
        p['gru_bih1'].reshape(1, -1), p['gru_bhh1'].reshape(1, -1))
    new_H = jnp.stack([h0T, h1T], axis=0)
    flat = y_tb.reshape(_NODENUM, _BATCH, _GRUH).transpose(1, 0, 2) \
        .reshape(_BATCH, _NODENUM * _GRUH)
    flat = jnp.pad(flat, ((0, 0), (0, 960)))
    cf_W1 = jnp.pad(p['cf_W1'], ((0, 960), (0, 0)))
    cf_out = _make_cf_kernel()(
        flat, cf_W1, p['cf_b1'].reshape(1, -1),
        p['cf_W2'], p['cf_b2'].reshape(1, -1),
        p['cf_W3'], p['cf_b3'].reshape(1, -1))

    ml = _make_mlp3_kernel(8, 128, 128, _SLID)
    rl = ml(cat, p['ml_W1'], p['ml_b1'].reshape(1, -1),
            p['ml_W2'], p['ml_b2'].reshape(1, -1),
            p['ml_W3'], p['ml_b3'].reshape(1, -1))
    recs = []
    for m in range(_MODENUM):
        r = _gat_layer(rl, p['r1_W'][m], p['r1_as'][m], p['r1_ad'][m],
                       p['r1_b'][m], 192, src3[m], dst3[m], _CFG_H6)
        r = _gat_layer(r, p['r2_W'][m], p['r2_as'][m], p['r2_ad'][m],
                       p['r2_b'][m], 128, src3[m], dst3[m], _CFG_R2)
        recs.append(r)
    rec_out = jnp.concatenate(recs, axis=0)
    return (cf_out, rec_out, new_H)


# speedup vs baseline: 24.5033x; 1.2527x over previous
import functools

import jax
import jax.numpy as jnp
from jax import lax
from jax.experimental import pallas as pl
from jax.experimental.pallas import tpu as pltpu
from jax.experimental.pallas import tpu_sc as plsc

_MODENUM = 2
_NODENUM = 2500
_BATCH = 4
_SLID = 128
_N = _NODENUM * _BATCH
_E = 320000
_EMB = 4
_HEADS = 6
_GRUH = 16

_C = 256
_SUB = 128
_NSUB = _C // _SUB
_EPAD = 327680
_NROW = _N + 8
_STRIPE = 624


def _leaky(x):
    return jnp.where(x >= 0, x, 0.2 * x)


@functools.lru_cache(maxsize=None)
def _make_edge_kernel(K, NH, HBMULT, esplit):
    vph = max(K // 16 // NH, 1)
    nchunks = _EPAD // 32 // _C if esplit else _EPAD // 16 // _C
    mesh = plsc.VectorSubcoreMesh(core_axis_name="c", subcore_axis_name="s")

    @functools.partial(
        pl.kernel,
        out_type=jax.ShapeDtypeStruct((2 * _N, 128), jnp.float32),
        mesh=mesh,
        scratch_types=[
            pltpu.VMEM((_NSUB, _SUB), jnp.int32),
            pltpu.VMEM((_NSUB, _SUB), jnp.int32),
            pltpu.VMEM((_NSUB, _SUB), jnp.int32),
            pltpu.VMEM((_C, 16), jnp.float32),
            pltpu.VMEM((_C, 16), jnp.float32),
            pltpu.VMEM((_C, 128), jnp.float32),
            pltpu.VMEM_SHARED((_NROW, 128), jnp.float32),
            pltpu.SemaphoreType.DMA,
        ],
        compiler_params=pltpu.CompilerParams(use_tc_tiling_on_sc=False),
    )
    def ek(src_hbm, dst_hbm, als_hbm, ald_hbm, tab_hbm,
           znum_hbm, num_out,
           src_v, dst_v, srcg_v, als_v, ald_v, h_v, sh_num, sem):
        c = lax.axis_index("c")
        s = lax.axis_index("s")

        @pl.when(s < 15)
        def _():
            pltpu.sync_copy(znum_hbm.at[pl.ds(0, _STRIPE)],
                            sh_num.at[pl.ds(s * _STRIPE, _STRIPE)])

        @pl.when(s == 15)
        def _():
            pltpu.sync_copy(znum_hbm, sh_num.at[pl.ds(15 * _STRIPE, 648)])

        plsc.subcore_barrier()

        if esplit:
            ck0 = (s * 2 + c) * nchunks
        else:
            ck0 = s * nchunks

        def chunk_body(i, carry0):
            ck = ck0 + i
            pltpu.sync_copy(src_hbm.at[ck], src_v)
            pltpu.sync_copy(dst_hbm.at[ck], dst_v)
            off = c * _NROW
            for j in range(_NSUB):
                for k in range(_SUB // 16):
                    sl = pl.ds(k * 16, 16)
                    srcg_v[j, sl] = src_v[j, sl] + off
            descs = []
            for j in range(_NSUB):
                dst_sl = pl.ds(j * _SUB, _SUB)
                descs.append(pltpu.async_copy(als_hbm.at[src_v.at[j]],
                                              als_v.at[dst_sl], sem))
                descs.append(pltpu.async_copy(ald_hbm.at[dst_v.at[j]],
                                              ald_v.at[dst_sl], sem))
                descs.append(pltpu.async_copy(tab_hbm.at[srcg_v.at[j]],
                                              h_v.at[dst_sl], sem))
            for dd in descs:
                dd.wait()

            def do_edges(hb):
                lanes = jnp.arange(16, dtype=jnp.int32)
                headmask = jnp.where((lanes >= hb) & (lanes < hb + NH),
                                     jnp.float32(1.0), jnp.float32(0.0))

                def edge_body(e, carry):
                    eerow = jnp.exp(_leaky(als_v[e] + ald_v[e]))
                    h_v[e, pl.ds(96, 16)] = eerow * headmask
                    for hh in range(NH):
                        m = eerow[hb + hh]
                        for jj in range(vph):
                            sl = pl.ds(16 * (hh * vph + jj), 16)
                            h_v[e, sl] = h_v[e, sl] * m
                    return carry

                lax.fori_loop(0, _C, edge_body, 0)

            if HBMULT == 0:
                do_edges(0)
            else:
                @pl.when(c == 0)
                def _():
                    do_edges(0)

                @pl.when(c == 1)
                def _():
                    do_edges(HBMULT)
            for j in range(_NSUB):
                src_sl = pl.ds(j * _SUB, _SUB)
                pltpu.sync_copy(h_v.at[src_sl],
                                sh_num.at[dst_v.at[j]], add=True)
            return carry0

        lax.fori_loop(0, nchunks, chunk_body, 0)
        plsc.subcore_barrier()
        ob = c * _N + s * _STRIPE

        @pl.when(s < 15)
        def _():
            pltpu.sync_copy(sh_num.at[pl.ds(s * _STRIPE, _STRIPE)],
                            num_out.at[pl.ds(ob, _STRIPE)])

        @pl.when(s == 15)
        def _():
            pltpu.sync_copy(sh_num.at[pl.ds(15 * _STRIPE, 640)],
                            num_out.at[pl.ds(c * _N + 15 * _STRIPE, 640)])

    return ek


_BN = 1000


@functools.lru_cache(maxsize=None)
def _make_proj_kernel(kin, kt):
    def body(x_ref, w_ref, as_ref, ad_ref, rex_ref,
             h_ref, als_ref, ald_ref, snum_ref, sden_ref):
        h = jnp.dot(x_ref[...], w_ref[...],
                    preferred_element_type=jnp.float32)
        h_ref[...] = h
        als = jnp.dot(h, as_ref[...], preferred_element_type=jnp.float32)
        ald = jnp.dot(h, ad_ref[...], preferred_element_type=jnp.float32)
        als_ref[...] = als
        ald_ref[...] = ald
        e = als + ald
        ee = jnp.exp(jnp.where(e >= 0, e, 0.2 * e))
        lanemask = (lax.broadcasted_iota(jnp.int32, (1, 16), 1) <
                    _n_heads(kt)).astype(jnp.float32)
        ee = ee * lanemask
        sden_ref[...] = ee
        snum_ref[...] = h * jnp.dot(ee, rex_ref[...],
                                    preferred_element_type=jnp.float32)

    grid = _N // _BN
    return pl.pallas_call(
        body,
        grid=(grid,),
        in_specs=[
            pl.BlockSpec((_BN, kin), lambda i: (i, 0)),
            pl.BlockSpec((kin, kt), lambda i: (0, 0)),
            pl.BlockSpec((kt, 16), lambda i: (0, 0)),
            pl.BlockSpec((kt, 16), lambda i: (0, 0)),
            pl.BlockSpec((16, kt), lambda i: (0, 0)),
        ],
        out_specs=[
            pl.BlockSpec((_BN, kt), lambda i: (i, 0)),
            pl.BlockSpec((_BN, 16), lambda i: (i, 0)),
            pl.BlockSpec((_BN, 16), lambda i: (i, 0)),
            pl.BlockSpec((_BN, kt), lambda i: (i, 0)),
            pl.BlockSpec((_BN, 16), lambda i: (i, 0)),
        ],
        out_shape=[
            jax.ShapeDtypeStruct((_N, kt), jnp.float32),
            jax.ShapeDtypeStruct((_N, 16), jnp.float32),
            jax.ShapeDtypeStruct((_N, 16), jnp.float32),
            jax.ShapeDtypeStruct((_N, kt), jnp.float32),
            jax.ShapeDtypeStruct((_N, 16), jnp.float32),
        ],
    )


def _n_heads(kt):
    return {192: 6, 4: 1, 128: 1}[kt]


@functools.lru_cache(maxsize=None)
def _make_finish_kernel(kt, esplit):
    heads = _n_heads(kt)

    def body(b0_ref, b1_ref, snum_ref, sden_ref, rex_ref, b_ref, o_ref):
        b0 = b0_ref[...]
        b1 = b1_ref[...]
        z = jnp.zeros((_BN, 16 - heads), jnp.float32)
        if esplit:
            num = b0[:, :kt] + b1[:, :kt]
            den16 = jnp.concatenate(
                [b0[:, 96:96 + 1] + b1[:, 96:96 + 1], z], axis=1)
        elif heads == 1:
            num = jnp.concatenate([b0[:, :kt // 2], b1[:, :kt // 2]], axis=1)
            den16 = jnp.concatenate([b0[:, 96:96 + 1], z], axis=1)
        else:
            nh = heads // 2
            num = jnp.concatenate([b0[:, :kt // 2], b1[:, :kt // 2]], axis=1)
            den16 = jnp.concatenate(
                [b0[:, 96:96 + nh], b1[:, 96 + nh:96 + 2 * nh], z], axis=1)
        den = jnp.dot(den16 + sden_ref[...], rex_ref[...],
                      preferred_element_type=jnp.float32)
        o_ref[...] = (num + snum_ref[...]) / (den + 1e-16) + b_ref[...]

    grid = _N // _BN
    return pl.pallas_call(
        body,
        grid=(grid,),
        in_specs=[
            pl.BlockSpec((_BN, 128), lambda i: (i, 0)),
            pl.BlockSpec((_BN, 128), lambda i: (i + _N // _BN, 0)),
            pl.BlockSpec((_BN, kt), lambda i: (i, 0)),
            pl.BlockSpec((_BN, 16), lambda i: (i, 0)),
            pl.BlockSpec((16, kt), lambda i: (0, 0)),
            pl.BlockSpec((1, kt), lambda i: (0, 0)),
        ],
        out_specs=pl.BlockSpec((_BN, kt), lambda i: (i, 0)),
        out_shape=jax.ShapeDtypeStruct((_N, kt), jnp.float32),
    )


@functools.lru_cache(maxsize=None)
def _make_mlp3_kernel(k0, k1, k2, k3):
    def body(x_ref, w1, b1, w2, b2, w3, b3, o_ref):
        z = jax.nn.relu(jnp.dot(x_ref[...], w1[...],
                                preferred_element_type=jnp.float32) + b1[...])
        z = jax.nn.relu(jnp.dot(z, w2[...],
                                preferred_element_type=jnp.float32) + b2[...])
        o_ref[...] = jnp.dot(z, w3[...],
                             preferred_element_type=jnp.float32) + b3[...]

    grid = _N // _BN
    return pl.pallas_call(
        body,
        grid=(grid,),
        in_specs=[
            pl.BlockSpec((_BN, k0), lambda i: (i, 0)),
            pl.BlockSpec((k0, k1), lambda i: (0, 0)),
            pl.BlockSpec((1, k1), lambda i: (0, 0)),
            pl.BlockSpec((k1, k2), lambda i: (0, 0)),
            pl.BlockSpec((1, k2), lambda i: (0, 0)),
            pl.BlockSpec((k2, k3), lambda i: (0, 0)),
            pl.BlockSpec((1, k3), lambda i: (0, 0)),
        ],
        out_specs=pl.BlockSpec((_BN, k3), lambda i: (i, 0)),
        out_shape=jax.ShapeDtypeStruct((_N, k3), jnp.float32),
    )


def _make_cat_kernel():
    def body(g0_ref, l0_ref, g1_ref, l1_ref, p0_ref, p1_ref, o_ref):
        c0 = jnp.concatenate([g0_ref[...], l0_ref[...]], axis=1)
        c1 = jnp.concatenate([g1_ref[...], l1_ref[...]], axis=1)
        o_ref[...] = (
            jnp.dot(c0, p0_ref[...], preferred_element_type=jnp.float32) +
            jnp.dot(c1, p1_ref[...], preferred_element_type=jnp.float32))

    grid = _N // _BN
    return pl.pallas_call(
        body,
        grid=(grid,),
        in_specs=[
            pl.BlockSpec((_BN, 4), lambda i: (i, 0)),
            pl.BlockSpec((_BN, 4), lambda i: (i, 0)),
            pl.BlockSpec((_BN, 4), lambda i: (i, 0)),
            pl.BlockSpec((_BN, 4), lambda i: (i, 0)),
            pl.BlockSpec((8, 8), lambda i: (0, 0)),
            pl.BlockSpec((8, 8), lambda i: (0, 0)),
        ],
        out_specs=pl.BlockSpec((_BN, 8), lambda i: (i, 0)),
        out_shape=jax.ShapeDtypeStruct((_N, 8), jnp.float32),
    )


def _make_gru_kernel():
    T = _NODENUM

    def step(gi, h, whhT, bhh):
        gh = jnp.dot(h, whhT, preferred_element_type=jnp.float32) + bhh
        r = jax.nn.sigmoid(gi[:, 0:16] + gh[:, 0:16])
        z = jax.nn.sigmoid(gi[:, 16:32] + gh[:, 16:32])
        nn = jnp.tanh(gi[:, 32:48] + r * gh[:, 32:48])
        return (1.0 - z) * nn + z * h

    def body(cat_ref, h00_ref, h10_ref, wih0, whh0, bih0, bhh0,
             wih1, whh1, bih1, bhh1, y_ref, ht0_ref, ht1_ref, gi_ref):
        def fill(i, carry):
            o = pl.multiple_of(i * 200, 8)
            gi_ref[pl.ds(o, 200), :] = jnp.dot(
                cat_ref[pl.ds(o, 200), :], wih0[...],
                preferred_element_type=jnp.float32) + bih0[...]
            return carry

        lax.fori_loop(0, (4 * T) // 200, fill, 0)

        def loop(t2, hs):
            h0, h1 = hs
            o = pl.multiple_of(t2 * 8, 8)
            g2 = gi_ref[pl.ds(o, 8), :]
            ys = []
            for half in range(2):
                gi = g2[4 * half:4 * half + 4, :]
                h0 = step(gi, h0, whh0[...], bhh0[...])
                gi1 = jnp.dot(h0, wih1[...],
                              preferred_element_type=jnp.float32) + bih1[...]
                h1 = step(gi1, h1, whh1[...], bhh1[...])
                ys.append(h1)
            y_ref[pl.ds(o, 8), :] = jnp.concatenate(ys, axis=0)
            return (h0, h1)

        h0, h1 = lax.fori_loop(0, T // 2, loop,
                               (h00_ref[...], h10_ref[...]))
        ht0_ref[...] = h0
        ht1_ref[...] = h1

    return pl.pallas_call(
        body,
        out_shape=[
            jax.ShapeDtypeStruct((4 * T, _GRUH), jnp.float32),
            jax.ShapeDtypeStruct((_BATCH, _GRUH), jnp.float32),
            jax.ShapeDtypeStruct((_BATCH, _GRUH), jnp.float32),
        ],
        scratch_shapes=[pltpu.VMEM((4 * T, 48), jnp.float32)],
    )


def _make_cf_kernel():
    KB = 2048
    NK = 40960 // KB

    def body(x_ref, w1_ref, b1, w2, b2, w3, b3, o_ref, acc_ref):
        k = pl.program_id(0)

        @pl.when(k == 0)
        def _():
            acc_ref[...] = jnp.zeros_like(acc_ref)

        acc_ref[...] += jnp.dot(x_ref[...], w1_ref[...],
                                preferred_element_type=jnp.float32)

        @pl.when(k == NK - 1)
        def _():
            z = jax.nn.relu(acc_ref[...] + b1[...])
            z = jax.nn.relu(jnp.dot(z, w2[...],
                                    preferred_element_type=jnp.float32)
                            + b2[...])
            o_ref[...] = jax.nn.sigmoid(
                jnp.dot(z, w3[...], preferred_element_type=jnp.float32)
                + b3[...])

    return pl.pallas_call(
        body,
        grid=(NK,),
        in_specs=[
            pl.BlockSpec((_BATCH, KB), lambda k: (0, k)),
            pl.BlockSpec((KB, 512), lambda k: (k, 0)),
            pl.BlockSpec((1, 512), lambda k: (0, 0)),
            pl.BlockSpec((512, 64), lambda k: (0, 0)),
            pl.BlockSpec((1, 64), lambda k: (0, 0)),
            pl.BlockSpec((64, 2), lambda k: (0, 0)),
            pl.BlockSpec((1, 2), lambda k: (0, 0)),
        ],
        out_specs=pl.BlockSpec((_BATCH, 2), lambda k: (0, 0)),
        out_shape=jax.ShapeDtypeStruct((_BATCH, 2), jnp.float32),
        scratch_shapes=[pltpu.VMEM((_BATCH, 512), jnp.float32)],
    )


def _att_mat(a, kt):
    heads, outc = a.shape
    m = jnp.zeros((kt, 16), jnp.float32)
    for h in range(heads):
        m = m.at[h * outc:(h + 1) * outc, h].set(a[h])
    return m


def _rex_mat(kt):
    heads = _n_heads(kt)
    outc = kt // heads
    m = jnp.zeros((16, kt), jnp.float32)
    for h in range(heads):
        m = m.at[h, h * outc:(h + 1) * outc].set(1.0)
    return m


def _pad_rows(a, cols=None):
    cpad = 0 if cols is None else cols - a.shape[1]
    return jnp.pad(a, ((0, _NROW - _N), (0, cpad)))


def _gat_layer(xh, W, a_s, a_d, b, kt, src3, dst3, sc_cfg):
    K, NH, HBMULT, esplit = sc_cfg
    heads = _n_heads(kt)
    proj = _make_proj_kernel(xh.shape[1], kt)
    rex = _rex_mat(kt)
    h, als, ald, snum, sden = proj(xh, W, _att_mat(a_s, kt),
                                   _att_mat(a_d, kt), rex)
    if esplit:
        tab0 = tab1 = _pad_rows(h, 128)
    else:
        half = kt // 2
        tab0 = _pad_rows(h[:, :half], 128)
        tab1 = _pad_rows(h[:, half:], 128)
    tab = jnp.concatenate([tab0, tab1], axis=0)
    ek = _make_edge_kernel(K, NH, HBMULT, esplit)
    znum = jnp.zeros((648, 128), jnp.float32)
    out = ek(src3, dst3, _pad_rows(als), _pad_rows(ald), tab, znum)
    fin = _make_finish_kernel(kt, esplit)
    return fin(out, out, snum, sden, rex, b.reshape(1, kt))


_CFG_H6 = (96, 3, 3, False)
_CFG_R2 = (64, 1, 0, False)
_CFG_G2 = (16, 1, 0, True)


def kernel(x, adj, H_, params):
    p = params
    src3 = []
    dst3 = []
    for m in range(_MODENUM):
        sp = jnp.full((_EPAD,), _N, jnp.int32).at[:_E].set(adj[m, 0])
        dp = jnp.full((_EPAD,), _N, jnp.int32).at[:_E].set(adj[m, 1])
        src3.append(sp.reshape(-1, _NSUB, _SUB))
        dst3.append(dp.reshape(-1, _NSUB, _SUB))

    nf = _make_mlp3_kernel(_SLID, 256, 32, _EMB)
    cats = _make_cat_kernel()
    g_list = []
    lf_list = []
    for m in range(_MODENUM):
        mt = x[m * _N:(m + 1) * _N]
        g = _gat_layer(mt, p['g1_W'][m], p['g1_as'][m], p['g1_ad'][m],
                       p['g1_b'][m], 192, src3[m], dst3[m], _CFG_H6)
        g = _gat_layer(g, p['g2_W'][m], p['g2_as'][m], p['g2_ad'][m],
                       p['g2_b'][m], 4, src3[m], dst3[m], _CFG_G2)
        lf = nf(mt, p['nf_W1'], p['nf_b1'].reshape(1, -1),
                p['nf_W2'], p['nf_b2'].reshape(1, -1),
                p['nf_W3'], p['nf_b3'].reshape(1, -1))
        g_list.append(g)
        lf_list.append(lf)
    cat = cats(g_list[0], lf_list[0], g_list[1], lf_list[1],
               p['catP'][0], p['catP'][1])

    cat_tb = cat.reshape(_BATCH, _NODENUM, 8).transpose(1, 0, 2) \
        .reshape(_BATCH * _NODENUM, 8)
    gru = _make_gru_kernel()
    y_tb, h0T, h1T = gru(
        cat_tb, H_[0], H_[1],
        p['gru_Wih0'].T, p['gru_Whh0'].T,
        p['gru_bih0'].reshape(1, -1), p['gru_bhh0'].reshape(1, -1),
        p['gru_Wih1'].T, p['gru_Whh1'].T,
        p['gru_bih1'].reshape(1, -1), p['gru_bhh1'].reshape(1, -1))
    new_H = jnp.stack([h0T, h1T], axis=0)
    flat = y_tb.reshape(_NODENUM, _BATCH, _GRUH).transpose(1, 0, 2) \
        .reshape(_BATCH, _NODENUM * _GRUH)
    flat = jnp.pad(flat, ((0, 0), (0, 960)))
    cf_W1 = jnp.pad(p['cf_W1'], ((0, 960), (0, 0)))
    cf_out = _make_cf_kernel()(
        flat, cf_W1, p['cf_b1'].reshape(1, -1),
        p['cf_W2'], p['cf_b2'].reshape(1, -1),
        p['cf_W3'], p['cf_b3'].reshape(1, -1))

    ml = _make_mlp3_kernel(8, 128, 128, _SLID)
    rl = ml(cat, p['ml_W1'], p['ml_b1'].reshape(1, -1),
            p['ml_W2'], p['ml_b2'].reshape(1, -1),
            p['ml_W3'], p['ml_b3'].reshape(1, -1))
    recs = []
    for m in range(_MODENUM):
        r = _gat_layer(rl, p['r1_W'][m], p['r1_as'][m], p['r1_ad'][m],
                       p['r1_b'][m], 192, src3[m], dst3[m], _CFG_H6)
        r = _gat_layer(r, p['r2_W'][m], p['r2_as'][m], p['r2_ad'][m],
                       p['r2_b'][m], 128, src3[m], dst3[m], _CFG_R2)
        recs.append(r)
    rec_out = jnp.concatenate(recs, axis=0)
    return (cf_out, rec_out, new_H)


# speedup vs baseline: 30.9921x; 1.2648x over previous
import functools

import jax
import jax.numpy as jnp
from jax import lax
from jax.experimental import pallas as pl
from jax.experimental.pallas import tpu as pltpu
from jax.experimental.pallas import tpu_sc as plsc

_MODENUM = 2
_NODENUM = 2500
_BATCH = 4
_SLID = 128
_N = _NODENUM * _BATCH
_E = 320000
_EMB = 4
_HEADS = 6
_GRUH = 16

_C = 256
_SUB = 128
_NSUB = _C // _SUB
_EPAD = 327680
_NROW = _N + 8
_STRIPE = 624


def _leaky(x):
    return jnp.where(x >= 0, x, 0.2 * x)


@functools.lru_cache(maxsize=None)
def _make_edge_kernel(K, NH, HBMULT, esplit):
    vph = max(K // 16 // NH, 1)
    nchunks = _EPAD // 32 // _C if esplit else _EPAD // 16 // _C
    mesh = plsc.VectorSubcoreMesh(core_axis_name="c", subcore_axis_name="s")

    @functools.partial(
        pl.kernel,
        out_type=jax.ShapeDtypeStruct((2 * _N, 128), jnp.float32),
        mesh=mesh,
        scratch_types=[
            pltpu.VMEM((_NSUB, _SUB), jnp.int32),
            pltpu.VMEM((_NSUB, _SUB), jnp.int32),
            pltpu.VMEM((_NSUB, _SUB), jnp.int32),
            pltpu.VMEM((_C, 16), jnp.float32),
            pltpu.VMEM((_C, 16), jnp.float32),
            pltpu.VMEM((_C, 128), jnp.float32),
            pltpu.VMEM_SHARED((_NROW, 128), jnp.float32),
            pltpu.SemaphoreType.DMA,
        ],
        compiler_params=pltpu.CompilerParams(use_tc_tiling_on_sc=False),
    )
    def ek(src_hbm, dst_hbm, als_hbm, ald_hbm, tab_hbm,
           znum_hbm, num_out,
           src_v, dst_v, srcg_v, als_v, ald_v, h_v, sh_num, sem):
        c = lax.axis_index("c")
        s = lax.axis_index("s")

        @pl.when(s < 15)
        def _():
            pltpu.sync_copy(znum_hbm.at[pl.ds(0, _STRIPE)],
                            sh_num.at[pl.ds(s * _STRIPE, _STRIPE)])

        @pl.when(s == 15)
        def _():
            pltpu.sync_copy(znum_hbm, sh_num.at[pl.ds(15 * _STRIPE, 648)])

        plsc.subcore_barrier()

        if esplit:
            ck0 = (s * 2 + c) * nchunks
        else:
            ck0 = s * nchunks

        def chunk_body(i, carry0):
            ck = ck0 + i
            pltpu.sync_copy(src_hbm.at[ck], src_v)
            pltpu.sync_copy(dst_hbm.at[ck], dst_v)
            off = c * _NROW
            for j in range(_NSUB):
                for k in range(_SUB // 16):
                    sl = pl.ds(k * 16, 16)
                    srcg_v[j, sl] = src_v[j, sl] + off
            descs = []
            for j in range(_NSUB):
                dst_sl = pl.ds(j * _SUB, _SUB)
                descs.append(pltpu.async_copy(als_hbm.at[src_v.at[j]],
                                              als_v.at[dst_sl], sem))
                descs.append(pltpu.async_copy(ald_hbm.at[dst_v.at[j]],
                                              ald_v.at[dst_sl], sem))
                descs.append(pltpu.async_copy(tab_hbm.at[srcg_v.at[j]],
                                              h_v.at[dst_sl], sem))
            for dd in descs:
                dd.wait()

            def do_edges(hb):
                lanes = jnp.arange(16, dtype=jnp.int32)
                headmask = jnp.where((lanes >= hb) & (lanes < hb + NH),
                                     jnp.float32(1.0), jnp.float32(0.0))

                @plsc.parallel_loop(0, _C, 1, unroll=4)
                def _(e):
                    eerow = jnp.exp(_leaky(als_v[e] + ald_v[e]))
                    h_v[e, pl.ds(96, 16)] = eerow * headmask
                    for hh in range(NH):
                        m = eerow[hb + hh]
                        for jj in range(vph):
                            sl = pl.ds(16 * (hh * vph + jj), 16)
                            h_v[e, sl] = h_v[e, sl] * m

            if HBMULT == 0:
                do_edges(0)
            else:
                @pl.when(c == 0)
                def _():
                    do_edges(0)

                @pl.when(c == 1)
                def _():
                    do_edges(HBMULT)
            for j in range(_NSUB):
                src_sl = pl.ds(j * _SUB, _SUB)
                pltpu.sync_copy(h_v.at[src_sl],
                                sh_num.at[dst_v.at[j]], add=True)
            return carry0

        lax.fori_loop(0, nchunks, chunk_body, 0)
        plsc.subcore_barrier()
        ob = c * _N + s * _STRIPE

        @pl.when(s < 15)
        def _():
            pltpu.sync_copy(sh_num.at[pl.ds(s * _STRIPE, _STRIPE)],
                            num_out.at[pl.ds(ob, _STRIPE)])

        @pl.when(s == 15)
        def _():
            pltpu.sync_copy(sh_num.at[pl.ds(15 * _STRIPE, 640)],
                            num_out.at[pl.ds(c * _N + 15 * _STRIPE, 640)])

    return ek


_BN = 1000


@functools.lru_cache(maxsize=None)
def _make_proj_kernel(kin, kt):
    def body(x_ref, w_ref, as_ref, ad_ref, rex_ref,
             h_ref, als_ref, ald_ref, snum_ref, sden_ref):
        h = jnp.dot(x_ref[...], w_ref[...],
                    preferred_element_type=jnp.float32)
        h_ref[...] = h
        als = jnp.dot(h, as_ref[...], preferred_element_type=jnp.float32)
        ald = jnp.dot(h, ad_ref[...], preferred_element_type=jnp.float32)
        als_ref[...] = als
        ald_ref[...] = ald
        e = als + ald
        ee = jnp.exp(jnp.where(e >= 0, e, 0.2 * e))
        lanemask = (lax.broadcasted_iota(jnp.int32, (1, 16), 1) <
                    _n_heads(kt)).astype(jnp.float32)
        ee = ee * lanemask
        sden_ref[...] = ee
        snum_ref[...] = h * jnp.dot(ee, rex_ref[...],
                                    preferred_element_type=jnp.float32)

    grid = _N // _BN
    return pl.pallas_call(
        body,
        grid=(grid,),
        in_specs=[
            pl.BlockSpec((_BN, kin), lambda i: (i, 0)),
            pl.BlockSpec((kin, kt), lambda i: (0, 0)),
            pl.BlockSpec((kt, 16), lambda i: (0, 0)),
            pl.BlockSpec((kt, 16), lambda i: (0, 0)),
            pl.BlockSpec((16, kt), lambda i: (0, 0)),
        ],
        out_specs=[
            pl.BlockSpec((_BN, kt), lambda i: (i, 0)),
            pl.BlockSpec((_BN, 16), lambda i: (i, 0)),
            pl.BlockSpec((_BN, 16), lambda i: (i, 0)),
            pl.BlockSpec((_BN, kt), lambda i: (i, 0)),
            pl.BlockSpec((_BN, 16), lambda i: (i, 0)),
        ],
        out_shape=[
            jax.ShapeDtypeStruct((_N, kt), jnp.float32),
            jax.ShapeDtypeStruct((_N, 16), jnp.float32),
            jax.ShapeDtypeStruct((_N, 16), jnp.float32),
            jax.ShapeDtypeStruct((_N, kt), jnp.float32),
            jax.ShapeDtypeStruct((_N, 16), jnp.float32),
        ],
    )


def _n_heads(kt):
    return {192: 6, 4: 1, 128: 1}[kt]


@functools.lru_cache(maxsize=None)
def _make_finish_kernel(kt, esplit):
    heads = _n_heads(kt)

    def body(b0_ref, b1_ref, snum_ref, sden_ref, rex_ref, b_ref, o_ref):
        b0 = b0_ref[...]
        b1 = b1_ref[...]
        z = jnp.zeros((_BN, 16 - heads), jnp.float32)
        if esplit:
            num = b0[:, :kt] + b1[:, :kt]
            den16 = jnp.concatenate(
                [b0[:, 96:96 + 1] + b1[:, 96:96 + 1], z], axis=1)
        elif heads == 1:
            num = jnp.concatenate([b0[:, :kt // 2], b1[:, :kt // 2]], axis=1)
            den16 = jnp.concatenate([b0[:, 96:96 + 1], z], axis=1)
        else:
            nh = heads // 2
            num = jnp.concatenate([b0[:, :kt // 2], b1[:, :kt // 2]], axis=1)
            den16 = jnp.concatenate(
                [b0[:, 96:96 + nh], b1[:, 96 + nh:96 + 2 * nh], z], axis=1)
        den = jnp.dot(den16 + sden_ref[...], rex_ref[...],
                      preferred_element_type=jnp.float32)
        o_ref[...] = (num + snum_ref[...]) / (den + 1e-16) + b_ref[...]

    grid = _N // _BN
    return pl.pallas_call(
        body,
        grid=(grid,),
        in_specs=[
            pl.BlockSpec((_BN, 128), lambda i: (i, 0)),
            pl.BlockSpec((_BN, 128), lambda i: (i + _N // _BN, 0)),
            pl.BlockSpec((_BN, kt), lambda i: (i, 0)),
            pl.BlockSpec((_BN, 16), lambda i: (i, 0)),
            pl.BlockSpec((16, kt), lambda i: (0, 0)),
            pl.BlockSpec((1, kt), lambda i: (0, 0)),
        ],
        out_specs=pl.BlockSpec((_BN, kt), lambda i: (i, 0)),
        out_shape=jax.ShapeDtypeStruct((_N, kt), jnp.float32),
    )


@functools.lru_cache(maxsize=None)
def _make_mlp3_kernel(k0, k1, k2, k3):
    def body(x_ref, w1, b1, w2, b2, w3, b3, o_ref):
        z = jax.nn.relu(jnp.dot(x_ref[...], w1[...],
                                preferred_element_type=jnp.float32) + b1[...])
        z = jax.nn.relu(jnp.dot(z, w2[...],
                                preferred_element_type=jnp.float32) + b2[...])
        o_ref[...] = jnp.dot(z, w3[...],
                             preferred_element_type=jnp.float32) + b3[...]

    grid = _N // _BN
    return pl.pallas_call(
        body,
        grid=(grid,),
        in_specs=[
            pl.BlockSpec((_BN, k0), lambda i: (i, 0)),
            pl.BlockSpec((k0, k1), lambda i: (0, 0)),
            pl.BlockSpec((1, k1), lambda i: (0, 0)),
            pl.BlockSpec((k1, k2), lambda i: (0, 0)),
            pl.BlockSpec((1, k2), lambda i: (0, 0)),
            pl.BlockSpec((k2, k3), lambda i: (0, 0)),
            pl.BlockSpec((1, k3), lambda i: (0, 0)),
        ],
        out_specs=pl.BlockSpec((_BN, k3), lambda i: (i, 0)),
        out_shape=jax.ShapeDtypeStruct((_N, k3), jnp.float32),
    )


def _make_cat_kernel():
    def body(g0_ref, l0_ref, g1_ref, l1_ref, p0_ref, p1_ref, o_ref):
        c0 = jnp.concatenate([g0_ref[...], l0_ref[...]], axis=1)
        c1 = jnp.concatenate([g1_ref[...], l1_ref[...]], axis=1)
        o_ref[...] = (
            jnp.dot(c0, p0_ref[...], preferred_element_type=jnp.float32) +
            jnp.dot(c1, p1_ref[...], preferred_element_type=jnp.float32))

    grid = _N // _BN
    return pl.pallas_call(
        body,
        grid=(grid,),
        in_specs=[
            pl.BlockSpec((_BN, 4), lambda i: (i, 0)),
            pl.BlockSpec((_BN, 4), lambda i: (i, 0)),
            pl.BlockSpec((_BN, 4), lambda i: (i, 0)),
            pl.BlockSpec((_BN, 4), lambda i: (i, 0)),
            pl.BlockSpec((8, 8), lambda i: (0, 0)),
            pl.BlockSpec((8, 8), lambda i: (0, 0)),
        ],
        out_specs=pl.BlockSpec((_BN, 8), lambda i: (i, 0)),
        out_shape=jax.ShapeDtypeStruct((_N, 8), jnp.float32),
    )


def _make_gru_kernel():
    T = _NODENUM

    def step(gi, h, whhT, bhh):
        gh = jnp.dot(h, whhT, preferred_element_type=jnp.float32) + bhh
        r = jax.nn.sigmoid(gi[:, 0:16] + gh[:, 0:16])
        z = jax.nn.sigmoid(gi[:, 16:32] + gh[:, 16:32])
        nn = jnp.tanh(gi[:, 32:48] + r * gh[:, 32:48])
        return (1.0 - z) * nn + z * h

    def body(cat_ref, h00_ref, h10_ref, wih0, whh0, bih0, bhh0,
             wih1, whh1, bih1, bhh1, y_ref, ht0_ref, ht1_ref, gi_ref):
        def fill(i, carry):
            o = pl.multiple_of(i * 200, 8)
            gi_ref[pl.ds(o, 200), :] = jnp.dot(
                cat_ref[pl.ds(o, 200), :], wih0[...],
                preferred_element_type=jnp.float32) + bih0[...]
            return carry

        lax.fori_loop(0, (4 * T) // 200, fill, 0)

        def loop(t2, hs):
            h0, h1 = hs
            o = pl.multiple_of(t2 * 8, 8)
            g2 = gi_ref[pl.ds(o, 8), :]
            ys = []
            for half in range(2):
                gi = g2[4 * half:4 * half + 4, :]
                h0 = step(gi, h0, whh0[...], bhh0[...])
                gi1 = jnp.dot(h0, wih1[...],
                              preferred_element_type=jnp.float32) + bih1[...]
                h1 = step(gi1, h1, whh1[...], bhh1[...])
                ys.append(h1)
            y_ref[pl.ds(o, 8), :] = jnp.concatenate(ys, axis=0)
            return (h0, h1)

        h0, h1 = lax.fori_loop(0, T // 2, loop,
                               (h00_ref[...], h10_ref[...]))
        ht0_ref[...] = h0
        ht1_ref[...] = h1

    return pl.pallas_call(
        body,
        out_shape=[
            jax.ShapeDtypeStruct((4 * T, _GRUH), jnp.float32),
            jax.ShapeDtypeStruct((_BATCH, _GRUH), jnp.float32),
            jax.ShapeDtypeStruct((_BATCH, _GRUH), jnp.float32),
        ],
        scratch_shapes=[pltpu.VMEM((4 * T, 48), jnp.float32)],
    )


def _make_cf_kernel():
    KB = 2048
    NK = 40960 // KB

    def body(x_ref, w1_ref, b1, w2, b2, w3, b3, o_ref, acc_ref):
        k = pl.program_id(0)

        @pl.when(k == 0)
        def _():
            acc_ref[...] = jnp.zeros_like(acc_ref)

        acc_ref[...] += jnp.dot(x_ref[...], w1_ref[...],
                                preferred_element_type=jnp.float32)

        @pl.when(k == NK - 1)
        def _():
            z = jax.nn.relu(acc_ref[...] + b1[...])
            z = jax.nn.relu(jnp.dot(z, w2[...],
                                    preferred_element_type=jnp.float32)
                            + b2[...])
            o_ref[...] = jax.nn.sigmoid(
                jnp.dot(z, w3[...], preferred_element_type=jnp.float32)
                + b3[...])

    return pl.pallas_call(
        body,
        grid=(NK,),
        in_specs=[
            pl.BlockSpec((_BATCH, KB), lambda k: (0, k)),
            pl.BlockSpec((KB, 512), lambda k: (k, 0)),
            pl.BlockSpec((1, 512), lambda k: (0, 0)),
            pl.BlockSpec((512, 64), lambda k: (0, 0)),
            pl.BlockSpec((1, 64), lambda k: (0, 0)),
            pl.BlockSpec((64, 2), lambda k: (0, 0)),
            pl.BlockSpec((1, 2), lambda k: (0, 0)),
        ],
        out_specs=pl.BlockSpec((_BATCH, 2), lambda k: (0, 0)),
        out_shape=jax.ShapeDtypeStruct((_BATCH, 2), jnp.float32),
        scratch_shapes=[pltpu.VMEM((_BATCH, 512), jnp.float32)],
    )


def _att_mat(a, kt):
    heads, outc = a.shape
    m = jnp.zeros((kt, 16), jnp.float32)
    for h in range(heads):
        m = m.at[h * outc:(h + 1) * outc, h].set(a[h])
    return m


def _rex_mat(kt):
    heads = _n_heads(kt)
    outc = kt // heads
    m = jnp.zeros((16, kt), jnp.float32)
    for h in range(heads):
        m = m.at[h, h * outc:(h + 1) * outc].set(1.0)
    return m


def _pad_rows(a, cols=None):
    cpad = 0 if cols is None else cols - a.shape[1]
    return jnp.pad(a, ((0, _NROW - _N), (0, cpad)))


def _gat_layer(xh, W, a_s, a_d, b, kt, src3, dst3, sc_cfg):
    K, NH, HBMULT, esplit = sc_cfg
    heads = _n_heads(kt)
    proj = _make_proj_kernel(xh.shape[1], kt)
    rex = _rex_mat(kt)
    h, als, ald, snum, sden = proj(xh, W, _att_mat(a_s, kt),
                                   _att_mat(a_d, kt), rex)
    if esplit:
        tab0 = tab1 = _pad_rows(h, 128)
    else:
        half = kt // 2
        tab0 = _pad_rows(h[:, :half], 128)
        tab1 = _pad_rows(h[:, half:], 128)
    tab = jnp.concatenate([tab0, tab1], axis=0)
    ek = _make_edge_kernel(K, NH, HBMULT, esplit)
    znum = jnp.zeros((648, 128), jnp.float32)
    out = ek(src3, dst3, _pad_rows(als), _pad_rows(ald), tab, znum)
    fin = _make_finish_kernel(kt, esplit)
    return fin(out, out, snum, sden, rex, b.reshape(1, kt))


_CFG_H6 = (96, 3, 3, False)
_CFG_R2 = (64, 1, 0, False)
_CFG_G2 = (16, 1, 0, True)


def kernel(x, adj, H_, params):
    p = params
    src3 = []
    dst3 = []
    for m in range(_MODENUM):
        sp = jnp.full((_EPAD,), _N, jnp.int32).at[:_E].set(adj[m, 0])
        dp = jnp.full((_EPAD,), _N, jnp.int32).at[:_E].set(adj[m, 1])
        src3.append(sp.reshape(-1, _NSUB, _SUB))
        dst3.append(dp.reshape(-1, _NSUB, _SUB))

    nf = _make_mlp3_kernel(_SLID, 256, 32, _EMB)
    cats = _make_cat_kernel()
    g_list = []
    lf_list = []
    for m in range(_MODENUM):
        mt = x[m * _N:(m + 1) * _N]
        g = _gat_layer(mt, p['g1_W'][m], p['g1_as'][m], p['g1_ad'][m],
                       p['g1_b'][m], 192, src3[m], dst3[m], _CFG_H6)
        g = _gat_layer(g, p['g2_W'][m], p['g2_as'][m], p['g2_ad'][m],
                       p['g2_b'][m], 4, src3[m], dst3[m], _CFG_G2)
        lf = nf(mt, p['nf_W1'], p['nf_b1'].reshape(1, -1),
                p['nf_W2'], p['nf_b2'].reshape(1, -1),
                p['nf_W3'], p['nf_b3'].reshape(1, -1))
        g_list.append(g)
        lf_list.append(lf)
    cat = cats(g_list[0], lf_list[0], g_list[1], lf_list[1],
               p['catP'][0], p['catP'][1])

    cat_tb = cat.reshape(_BATCH, _NODENUM, 8).transpose(1, 0, 2) \
        .reshape(_BATCH * _NODENUM, 8)
    gru = _make_gru_kernel()
    y_tb, h0T, h1T = gru(
        cat_tb, H_[0], H_[1],
        p['gru_Wih0'].T, p['gru_Whh0'].T,
        p['gru_bih0'].reshape(1, -1), p['gru_bhh0'].reshape(1, -1),
        p['gru_Wih1'].T, p['gru_Whh1'].T,
        p['gru_bih1'].reshape(1, -1), p['gru_bhh1'].reshape(1, -1))
    new_H = jnp.stack([h0T, h1T], axis=0)
    flat = y_tb.reshape(_NODENUM, _BATCH, _GRUH).transpose(1, 0, 2) \
        .reshape(_BATCH, _NODENUM * _GRUH)
    flat = jnp.pad(flat, ((0, 0), (0, 960)))
    cf_W1 = jnp.pad(p['cf_W1'], ((0, 960), (0, 0)))
    cf_out = _make_cf_kernel()(
        flat, cf_W1, p['cf_b1'].reshape(1, -1),
        p['cf_W2'], p['cf_b2'].reshape(1, -1),
        p['cf_W3'], p['cf_b3'].reshape(1, -1))

    ml = _make_mlp3_kernel(8, 128, 128, _SLID)
    rl = ml(cat, p['ml_W1'], p['ml_b1'].reshape(1, -1),
            p['ml_W2'], p['ml_b2'].reshape(1, -1),
            p['ml_W3'], p['ml_b3'].reshape(1, -1))
    recs = []
    for m in range(_MODENUM):
        r = _gat_layer(rl, p['r1_W'][m], p['r1_as'][m], p['r1_ad'][m],
                       p['r1_b'][m], 192, src3[m], dst3[m], _CFG_H6)
        r = _gat_layer(r, p['r2_W'][m], p['r2_as'][m], p['r2_ad'][m],
                       p['r2_b'][m], 128, src3[m], dst3[m], _CFG_R2)
        recs.append(r)
    rec_out = jnp.concatenate(recs, axis=0)
    return (cf_out, rec_out, new_H)
